# baseline (speedup 1.0000x reference)
import functools

import jax
import jax.numpy as jnp
import numpy as np
from jax.experimental import pallas as pl

_C = 32
_MIX = np.float32(0.01) * np.float32(1.0) / np.float32(32)
_KEEP = np.float32(1.0) - np.float32(0.01)
_TINY = np.finfo(np.float32).tiny


def _threefry_bits(flat_u32):
    u32 = jnp.uint32
    ks0 = u32(0)
    ks1 = u32(42)
    ks2 = ks0 ^ ks1 ^ u32(0x1BD11BDA)
    ks = (ks0, ks1, ks2)

    def rotl(v, d):
        return (v << u32(d)) | (v >> u32(32 - d))

    x0 = jnp.zeros_like(flat_u32) + ks0
    x1 = flat_u32 + ks1
    rots = ((13, 15, 26, 6), (17, 29, 16, 24))
    sched = ((1, 2, 1), (2, 0, 2), (0, 1, 3), (1, 2, 4), (2, 0, 5))
    for g in range(5):
        for r in rots[g % 2]:
            x0 = x0 + x1
            x1 = rotl(x1, r)
            x1 = x1 ^ x0
        a, b, inc = sched[g]
        x0 = x0 + ks[a]
        x1 = x1 + ks[b] + u32(inc)
    return x0 ^ x1


def _body(x_ref, o_ref, *, block_rows):
    x = x_ref[:, :]

    r_iota = jax.lax.broadcasted_iota(jnp.int32, x.shape, 0)
    c_iota = jax.lax.broadcasted_iota(jnp.int32, x.shape, 1)
    base = pl.program_id(0) * (block_rows * 1024)
    flat = (base + r_iota * 1024 + c_iota).astype(jnp.uint32)
    bits = _threefry_bits(flat)
    fb = (bits >> jnp.uint32(9)) | jnp.uint32(0x3F800000)
    u = jax.lax.bitcast_convert_type(fb, jnp.float32) - jnp.float32(1.0)
    tiny = jnp.float32(_TINY)
    u = jnp.maximum(tiny, u * (jnp.float32(1.0) - tiny) + tiny)
    g = -jnp.log(-jnp.log(u))

    x3 = x.reshape(block_rows, _C, _C)
    mx = jnp.max(x3, axis=-1, keepdims=True)
    e = jnp.exp(x3 - mx)
    s = jnp.sum(e, axis=-1, keepdims=True)
    p = jnp.float32(_MIX) + jnp.float32(_KEEP) * (e / s)

    v = g.reshape(block_rows, _C, _C) + jnp.log(p)
    m = jnp.max(v, axis=-1, keepdims=True)
    kio = jax.lax.broadcasted_iota(jnp.int32, v.shape, 2)
    cand = jnp.where(v == m, kio, jnp.int32(_C))
    idx = jnp.min(cand, axis=-1, keepdims=True)
    onehot = (kio == idx).astype(jnp.float32)

    o_ref[:, :] = ((onehot + p) - p).reshape(block_rows, _C * _C)


@jax.jit
def kernel(logits):
    rows, cols = logits.shape
    block_rows = 512
    grid = (rows // block_rows,)
    out = pl.pallas_call(
        functools.partial(_body, block_rows=block_rows),
        out_shape=jax.ShapeDtypeStruct((rows, cols), jnp.float32),
        grid=grid,
        in_specs=[pl.BlockSpec((block_rows, cols), lambda i: (i, 0))],
        out_specs=pl.BlockSpec((block_rows, cols), lambda i: (i, 0)),
    )(logits)
    return out.reshape(rows, _C, _C)


# speedup vs baseline: 1.2885x; 1.2885x over previous
import functools

import jax
import jax.numpy as jnp
import numpy as np
from jax.experimental import pallas as pl
from jax.experimental.pallas import tpu as pltpu

_C = 32
_MIX = np.float32(0.01) * np.float32(1.0) / np.float32(32)
_KEEP = np.float32(1.0) - np.float32(0.01)
_TINY = np.finfo(np.float32).tiny


def _threefry_bits(flat_u32):
    u32 = jnp.uint32
    ks0 = u32(0)
    ks1 = u32(42)
    ks2 = ks0 ^ ks1 ^ u32(0x1BD11BDA)
    ks = (ks0, ks1, ks2)

    def rotl(v, d):
        return (v << u32(d)) | (v >> u32(32 - d))

    x0 = jnp.zeros_like(flat_u32) + ks0
    x1 = flat_u32 + ks1
    rots = ((13, 15, 26, 6), (17, 29, 16, 24))
    sched = ((1, 2, 1), (2, 0, 2), (0, 1, 3), (1, 2, 4), (2, 0, 5))
    for g in range(5):
        for r in rots[g % 2]:
            x0 = x0 + x1
            x1 = rotl(x1, r)
            x1 = x1 ^ x0
        a, b, inc = sched[g]
        x0 = x0 + ks[a]
        x1 = x1 + ks[b] + u32(inc)
    return x0 ^ x1


def _seg_reduce(x, c_iota, op):
    for o in (1, 2, 4, 8, 16):
        fwd = pltpu.roll(x, x.shape[1] - o, 1)
        bwd = pltpu.roll(x, o, 1)
        partner = jnp.where((c_iota & o) == 0, fwd, bwd)
        x = op(x, partner)
    return x


def _body(x_ref, o_ref, *, block_rows):
    x = x_ref[:, :]

    r_iota = jax.lax.broadcasted_iota(jnp.int32, x.shape, 0)
    c_iota = jax.lax.broadcasted_iota(jnp.int32, x.shape, 1)
    base = pl.program_id(0) * (block_rows * 1024)
    flat = (base + r_iota * 1024 + c_iota).astype(jnp.uint32)
    bits = _threefry_bits(flat)
    fb = (bits >> jnp.uint32(9)) | jnp.uint32(0x3F800000)
    u = jax.lax.bitcast_convert_type(fb, jnp.float32) - jnp.float32(1.0)
    tiny = jnp.float32(_TINY)
    u = jnp.maximum(tiny, u * (jnp.float32(1.0) - tiny) + tiny)
    w = -jnp.log(u)

    e = jnp.exp(x)
    s = _seg_reduce(e, c_iota, jnp.add)
    d = jnp.float32(_MIX) * s + jnp.float32(_KEEP) * e

    race = w / d
    m = _seg_reduce(race, c_iota, jnp.minimum)
    o_ref[:, :] = (race == m).astype(jnp.float32)


@jax.jit
def kernel(logits):
    rows, cols = logits.shape
    block_rows = 512
    grid = (rows // block_rows,)
    out = pl.pallas_call(
        functools.partial(_body, block_rows=block_rows),
        out_shape=jax.ShapeDtypeStruct((rows, cols), jnp.float32),
        grid=grid,
        in_specs=[pl.BlockSpec((block_rows, cols), lambda i: (i, 0))],
        out_specs=pl.BlockSpec((block_rows, cols), lambda i: (i, 0)),
    )(logits)
    return out.reshape(rows, _C, _C)


# speedup vs baseline: 1.5435x; 1.1978x over previous
import functools

import jax
import jax.numpy as jnp
import numpy as np
from jax.experimental import pallas as pl
from jax.experimental.pallas import tpu as pltpu

_C = 32
_MIX = np.float32(0.01) * np.float32(1.0) / np.float32(32)
_KEEP = np.float32(1.0) - np.float32(0.01)
_TINY = np.finfo(np.float32).tiny


def _threefry_bits(flat_u32):
    u32 = jnp.uint32
    ks0 = u32(0)
    ks1 = u32(42)
    ks2 = ks0 ^ ks1 ^ u32(0x1BD11BDA)
    ks = (ks0, ks1, ks2)

    def rotl(v, d):
        return (v << u32(d)) | (v >> u32(32 - d))

    x0 = jnp.zeros_like(flat_u32) + ks0
    x1 = flat_u32 + ks1
    rots = ((13, 15, 26, 6), (17, 29, 16, 24))
    sched = ((1, 2, 1), (2, 0, 2), (0, 1, 3), (1, 2, 4), (2, 0, 5))
    for g in range(5):
        for r in rots[g % 2]:
            x0 = x0 + x1
            x1 = rotl(x1, r)
            x1 = x1 ^ x0
        a, b, inc = sched[g]
        x0 = x0 + ks[a]
        x1 = x1 + ks[b] + u32(inc)
    return x0 ^ x1


def _body(x_ref, o_ref, b_ref, *, block_rows):
    @pl.when(pl.program_id(0) == 0)
    def _():
        br = jax.lax.broadcasted_iota(jnp.int32, b_ref.shape, 0)
        bc = jax.lax.broadcasted_iota(jnp.int32, b_ref.shape, 1)
        seg = ((br >> 5) == (bc >> 5)).astype(jnp.float32) * jnp.float32(_MIX)
        eye = (br == bc).astype(jnp.float32) * jnp.float32(_KEEP)
        b_ref[:, :] = seg + eye

    x = x_ref[:, :]

    r_iota = jax.lax.broadcasted_iota(jnp.int32, x.shape, 0)
    c_iota = jax.lax.broadcasted_iota(jnp.int32, x.shape, 1)
    base = pl.program_id(0) * (block_rows * 1024)
    flat = (base + r_iota * 1024 + c_iota).astype(jnp.uint32)
    bits = _threefry_bits(flat)
    fb = (bits >> jnp.uint32(9)) | jnp.uint32(0x3F800000)
    u = jax.lax.bitcast_convert_type(fb, jnp.float32) - jnp.float32(1.0)
    tiny = jnp.float32(_TINY)
    u = jnp.maximum(tiny, u * (jnp.float32(1.0) - tiny) + tiny)
    w = -jnp.log(u)

    e = jnp.exp(x)
    d = jax.lax.dot_general(e, b_ref[:, :], (((1,), (0,)), ((), ())),
                            preferred_element_type=jnp.float32)

    race = (w / d).reshape(block_rows, _C, _C)
    m = jnp.min(race, axis=-1, keepdims=True)
    o_ref[:, :, :] = (race == m).astype(jnp.float32)


@jax.jit
def kernel(logits):
    rows, cols = logits.shape
    block_rows = 512
    grid = (rows // block_rows,)
    return pl.pallas_call(
        functools.partial(_body, block_rows=block_rows),
        out_shape=jax.ShapeDtypeStruct((rows, _C, _C), jnp.float32),
        grid=grid,
        in_specs=[pl.BlockSpec((block_rows, cols), lambda i: (i, 0))],
        out_specs=pl.BlockSpec((block_rows, _C, _C), lambda i: (i, 0, 0)),
        scratch_shapes=[pltpu.VMEM((cols, cols), jnp.float32)],
    )(logits)
